# Initial kernel scaffold; baseline (speedup 1.0000x reference)
import jax
import jax.numpy as jnp
from jax.experimental import pallas as pl


def kernel(x, adj_t, W1, b1, g1, bt1, W2, b2, g2, bt2, W3, b3):
    raise NotImplementedError("write your pallas kernel here")


# speedup vs baseline: 17.4724x; 17.4724x over previous
import functools

import jax
import jax.numpy as jnp
from jax import lax
from jax.experimental import pallas as pl
from jax.experimental.pallas import tpu as pltpu
from jax.experimental.pallas import tpu_sc as plsc

N = 10000
E = 320000
NC, NS = 2, 16
NW = NC * NS
EPW = E // NW
CHUNK = 80
NCHUNK = EPW // CHUNK
RPT = N // NS

_mesh = plsc.VectorSubcoreMesh(core_axis_name="c", subcore_axis_name="s")
_sc_params = pltpu.CompilerParams(use_tc_tiling_on_sc=False)


def _make_agg(D):
    @functools.partial(
        pl.kernel,
        out_type=jax.ShapeDtypeStruct((NC, N, D), jnp.float32),
        mesh=_mesh,
        scratch_types=[
            pltpu.VMEM((NCHUNK, CHUNK), jnp.int32),
            pltpu.VMEM((NCHUNK, CHUNK), jnp.int32),
            pltpu.VMEM((CHUNK, D), jnp.float32),
            pltpu.VMEM_SHARED((N, D), jnp.float32),
            pltpu.SemaphoreType.DMA,
        ],
        compiler_params=_sc_params,
    )
    def agg(g_hbm, src_hbm, dst_hbm, zeros_hbm, out_hbm,
            src_v, dst_v, rows_v, acc, sem):
        cid = lax.axis_index("c")
        sid = lax.axis_index("s")
        wid = cid * NS + sid
        pltpu.sync_copy(zeros_hbm, acc.at[pl.ds(sid * RPT, RPT)])
        pltpu.sync_copy(src_hbm.at[wid], src_v)
        pltpu.sync_copy(dst_hbm.at[wid], dst_v)
        plsc.subcore_barrier()

        @pl.loop(0, NCHUNK)
        def _(j):
            pltpu.async_copy(g_hbm.at[src_v.at[j]], rows_v, sem).wait()
            pltpu.sync_copy(rows_v, acc.at[dst_v.at[j]], add=True)

        plsc.subcore_barrier()
        pltpu.sync_copy(acc.at[pl.ds(sid * RPT, RPT)],
                        out_hbm.at[cid].at[pl.ds(sid * RPT, RPT)])

    return agg


_agg128 = _make_agg(128)
_agg48 = _make_agg(48)

DEGW = 16


@functools.partial(
    pl.kernel,
    out_type=jax.ShapeDtypeStruct((NC, N, DEGW), jnp.float32),
    mesh=_mesh,
    scratch_types=[
        pltpu.VMEM((NCHUNK, CHUNK), jnp.int32),
        pltpu.VMEM((CHUNK, DEGW), jnp.float32),
        pltpu.VMEM_SHARED((N, DEGW), jnp.float32),
    ],
    compiler_params=_sc_params,
)
def _deg(dst_hbm, ones_hbm, zeros_hbm, out_hbm, dst_v, ones_v, acc):
    cid = lax.axis_index("c")
    sid = lax.axis_index("s")
    wid = cid * NS + sid
    pltpu.sync_copy(zeros_hbm, acc.at[pl.ds(sid * RPT, RPT)])
    pltpu.sync_copy(dst_hbm.at[wid], dst_v)
    pltpu.sync_copy(ones_hbm, ones_v)
    plsc.subcore_barrier()

    @pl.loop(0, NCHUNK)
    def _(j):
        pltpu.sync_copy(ones_v, acc.at[dst_v.at[j]], add=True)

    plsc.subcore_barrier()
    pltpu.sync_copy(acc.at[pl.ds(sid * RPT, RPT)],
                    out_hbm.at[cid].at[pl.ds(sid * RPT, RPT)])


_DOT = dict(preferred_element_type=jnp.float32, precision=lax.Precision.HIGHEST)


def _tc(fn, out_shape, *args):
    return pl.pallas_call(
        fn, out_shape=jax.ShapeDtypeStruct(out_shape, jnp.float32))(*args)


def _first_kernel(degp_ref, x_ref, w1_ref, g1_ref, dinv_ref):
    deg = degp_ref[0, :, 0:1] + degp_ref[1, :, 0:1] + 1.0
    dinv = lax.rsqrt(deg)
    dinv_ref[...] = dinv
    g1_ref[...] = jnp.dot(x_ref[...], w1_ref[...], **_DOT) * dinv


def _mid_kernel(parts_ref, g_ref, dinv_ref, b_ref, gam_ref, bet_ref, w_ref,
                gn_ref):
    dinv = dinv_ref[...]
    t = dinv * (parts_ref[0] + parts_ref[1] + g_ref[...]) + b_ref[...]
    mean = jnp.mean(t, axis=0, keepdims=True)
    xc = t - mean
    var = jnp.mean(xc * xc, axis=0, keepdims=True)
    y = gam_ref[...] * (xc / jnp.sqrt(var + 1e-5)) + bet_ref[...]
    y = jnp.maximum(y, 0.0)
    gn_ref[...] = jnp.dot(y, w_ref[...], **_DOT) * dinv


def _last_kernel(parts_ref, g_ref, dinv_ref, b_ref, out_ref):
    t = dinv_ref[...] * (parts_ref[0] + parts_ref[1] + g_ref[...])
    t = t[:, 0:40] + b_ref[...]
    m = jnp.max(t, axis=1, keepdims=True)
    s = jnp.sum(jnp.exp(t - m), axis=1, keepdims=True)
    out_ref[...] = t - (m + jnp.log(s))


def kernel(x, adj_t, W1, b1, g1, bt1, W2, b2, g2, bt2, W3, b3):
    src = adj_t[0].astype(jnp.int32).reshape(NW, NCHUNK, CHUNK)
    dst = adj_t[1].astype(jnp.int32).reshape(NW, NCHUNK, CHUNK)
    zeros128 = jnp.zeros((RPT, 128), jnp.float32)
    zeros48 = jnp.zeros((RPT, 48), jnp.float32)
    zerosdw = jnp.zeros((RPT, DEGW), jnp.float32)
    ones_rows = jnp.ones((CHUNK, DEGW), jnp.float32)
    W3p = jnp.pad(W3, ((0, 0), (0, 8)))

    degp = _deg(dst, ones_rows, zerosdw)
    h1, dinv = pl.pallas_call(
        _first_kernel,
        out_shape=(jax.ShapeDtypeStruct((N, 128), jnp.float32),
                   jax.ShapeDtypeStruct((N, 1), jnp.float32)),
    )(degp, x, W1)

    p1 = _agg128(h1, src, dst, zeros128)
    h2 = _tc(_mid_kernel, (N, 128), p1, h1, dinv, b1.reshape(1, 128),
             g1.reshape(1, 128), bt1.reshape(1, 128), W2)

    p2 = _agg128(h2, src, dst, zeros128)
    h3 = _tc(_mid_kernel, (N, 48), p2, h2, dinv, b2.reshape(1, 128),
             g2.reshape(1, 128), bt2.reshape(1, 128), W3p)

    p3 = _agg48(h3, src, dst, zeros48)
    return _tc(_last_kernel, (N, 40), p3, h3, dinv, b3.reshape(1, 40))
